# baseline (speedup 1.0000x reference)
import jax
import jax.numpy as jnp
from jax.experimental import pallas as pl


BB = 16


def _concat_body(xyz_ref, pts_ref, out_ref):
    for i in range(BB):
        out_ref[i, 0:3, 0, :] = xyz_ref[0:3, i, :]
        out_ref[i, 3:, 0, :] = pts_ref[i]


def kernel(xyz, points):
    B, N, C = xyz.shape
    D = points.shape[-1]
    F = C + D
    xyz_t = jnp.transpose(xyz, (2, 0, 1))
    pts_t = jnp.transpose(points, (0, 2, 1))
    out_t = pl.pallas_call(
        _concat_body,
        grid=(B // BB, 4),
        in_specs=[
            pl.BlockSpec((C, BB, N // 4), lambda b, n: (0, b, n)),
            pl.BlockSpec((BB, D, N // 4), lambda b, n: (b, 0, n)),
        ],
        out_specs=pl.BlockSpec((BB, F, 1, N // 4), lambda b, n: (b, 0, 0, n)),
        out_shape=jax.ShapeDtypeStruct((B, F, 1, N), xyz.dtype),
    )(xyz_t, pts_t)
    new_xyz = jnp.zeros((B, 1, C), dtype=xyz.dtype)
    return new_xyz, jnp.transpose(out_t, (0, 2, 3, 1))
